# baseline (speedup 1.0000x reference)
import functools

import jax
import jax.numpy as jnp
import numpy as np
from jax.experimental import pallas as pl
from jax.experimental.pallas import tpu as pltpu
from jax.sharding import Mesh, PartitionSpec as P

if hasattr(jax, "shard_map"):
    _shard_map = functools.partial(jax.shard_map, check_vma=False)
else:
    from jax.experimental.shard_map import shard_map as _shard_map_exp

    _shard_map = functools.partial(_shard_map_exp, check_rep=False)


def _mm1_kernel(x_ref, w1_ref, s_ref):
    acc = jnp.dot(x_ref[...], w1_ref[...], preferred_element_type=jnp.float32)
    s_ref[...] = acc.astype(jnp.bfloat16)


def _mm2_kernel(w_ref, s_ref, o_ref):
    wt = w_ref[...].astype(jnp.bfloat16)
    acc = jnp.dot(wt, s_ref[...], preferred_element_type=jnp.float32)
    o_ref[...] = jnp.where(acc >= 0, acc, 0.1 * acc)


def _gcn_shard(x, w_shard, W1):
    n, nfeat = x.shape
    rows = w_shard.shape[0]
    nhid = W1.shape[1]

    support = pl.pallas_call(
        _mm1_kernel,
        out_shape=jax.ShapeDtypeStruct((n, nhid), jnp.bfloat16),
    )(x, W1)

    bm = next((b for b in (400, 200, 100, 80, 40, 8) if rows % b == 0), None)
    if bm is None:
        acc = jnp.dot(
            w_shard.astype(jnp.bfloat16), support,
            preferred_element_type=jnp.float32,
        )
        return jnp.where(acc >= 0, acc, 0.1 * acc)

    return pl.pallas_call(
        _mm2_kernel,
        grid=(rows // bm,),
        in_specs=[
            pl.BlockSpec((bm, n), lambda i: (i, 0)),
            pl.BlockSpec((n, nhid), lambda i: (0, 0)),
        ],
        out_specs=pl.BlockSpec((bm, nhid), lambda i: (i, 0)),
        out_shape=jax.ShapeDtypeStruct((rows, nhid), jnp.float32),
        compiler_params=pltpu.CompilerParams(
            dimension_semantics=("parallel",),
        ),
    )(w_shard, support)


def kernel(x, w, W1):
    n = w.shape[0]
    devs = jax.devices()
    ndev = len(devs)
    if ndev > 1 and n % ndev == 0:
        mesh = Mesh(np.array(devs), ("d",))
        fn = _shard_map(
            _gcn_shard,
            mesh=mesh,
            in_specs=(P(), P("d", None), P()),
            out_specs=P("d", None),
        )
        return fn(x, w, W1)
    return _gcn_shard(x, w, W1)


# speedup vs baseline: 5.2571x; 5.2571x over previous
import jax
import jax.numpy as jnp
from jax.experimental import pallas as pl
from jax.experimental.pallas import tpu as pltpu

_BM = 400


def _gcn_kernel(x_ref, w1_ref, w_ref, o_ref, s_ref):
    @pl.when(pl.program_id(0) == 0)
    def _():
        acc = jnp.dot(
            x_ref[...], w1_ref[...], preferred_element_type=jnp.float32
        )
        s_ref[...] = acc.astype(jnp.bfloat16)

    wt = w_ref[...].astype(jnp.bfloat16)
    acc = jnp.dot(wt, s_ref[...], preferred_element_type=jnp.float32)
    o_ref[...] = jnp.where(acc >= 0, acc, 0.1 * acc)


def kernel(x, w, W1):
    n, nfeat = x.shape
    nhid = W1.shape[1]

    return pl.pallas_call(
        _gcn_kernel,
        grid=(n // _BM,),
        in_specs=[
            pl.BlockSpec((n, nfeat), lambda i: (0, 0)),
            pl.BlockSpec((nfeat, nhid), lambda i: (0, 0)),
            pl.BlockSpec((_BM, n), lambda i: (i, 0)),
        ],
        out_specs=pl.BlockSpec((_BM, nhid), lambda i: (i, 0)),
        out_shape=jax.ShapeDtypeStruct((n, nhid), jnp.float32),
        scratch_shapes=[pltpu.VMEM((n, nhid), jnp.bfloat16)],
    )(x, W1, w)


# speedup vs baseline: 5.2852x; 1.0053x over previous
import jax
import jax.numpy as jnp
from jax.experimental import pallas as pl
from jax.experimental.pallas import tpu as pltpu

_BM = 400


def _gcn_kernel(x_ref, w1_ref, w_ref, o_ref, s_ref):
    @pl.when(pl.program_id(0) == 0)
    def _():
        s_ref[...] = jnp.dot(
            x_ref[...], w1_ref[...], preferred_element_type=jnp.float32
        )

    acc = jax.lax.dot_general(
        w_ref[...],
        s_ref[...],
        (((1,), (0,)), ((), ())),
        precision=jax.lax.Precision.DEFAULT,
        preferred_element_type=jnp.float32,
    )
    o_ref[...] = jnp.where(acc >= 0, acc, 0.1 * acc)


def kernel(x, w, W1):
    n, nfeat = x.shape
    nhid = W1.shape[1]

    return pl.pallas_call(
        _gcn_kernel,
        grid=(n // _BM,),
        in_specs=[
            pl.BlockSpec((n, nfeat), lambda i: (0, 0)),
            pl.BlockSpec((nfeat, nhid), lambda i: (0, 0)),
            pl.BlockSpec((_BM, n), lambda i: (i, 0)),
        ],
        out_specs=pl.BlockSpec((_BM, nhid), lambda i: (i, 0)),
        out_shape=jax.ShapeDtypeStruct((n, nhid), jnp.float32),
        scratch_shapes=[pltpu.VMEM((n, nhid), jnp.float32)],
    )(x, W1, w)
